# baseline (speedup 1.0000x reference)
import functools

import jax
import jax.numpy as jnp
from jax import lax
from jax.experimental import pallas as pl
from jax.experimental.pallas import tpu as pltpu
from jax.experimental.pallas import tpu_sc as plsc

A1 = 0.3981
A2 = 4.4211
LANES = 16
TILE = 128
N_WORKERS = 32
CHUNK_TILES = 39
CHUNK = CHUNK_TILES * TILE
N_CHUNKS = 10


def _pow6(x):
    x2 = x * x
    return x2 * x2 * x2


def _damp16(idx, damp_tbl):
    return lax.gather(
        damp_tbl, idx[:, None],
        lax.GatherDimensionNumbers(offset_dims=(),
                                   collapsed_slice_dims=(0,),
                                   start_index_map=(0,)),
        slice_sizes=(1,),
        mode=lax.GatherScatterMode.PROMISE_IN_BOUNDS)


def _tec_body(n_extra_workers, species_hbm, dist_hbm, table_hbm, out_hbm,
              tbl_v, sa, sb, da, db, oa, ob, se_v, de_v, oe_v, sems):
    nc = 2
    wid = lax.axis_index("s") * nc + lax.axis_index("c")
    start = (wid * (N_CHUNKS * CHUNK_TILES)
             + jnp.minimum(wid, n_extra_workers)) * TILE
    bufs = [(sa, da, oa), (sb, db, ob)]

    pltpu.sync_copy(table_hbm, tbl_v)
    damp_tbl = _pow6(A1 * tbl_v[...] + A2)

    def in_copies(c, b):
        off = start + c * CHUNK
        s_v, d_v, _ = bufs[b]
        return [
            pltpu.make_async_copy(species_hbm.at[:, pl.ds(off, CHUNK)],
                                  s_v, sems.at[0, b]),
            pltpu.make_async_copy(dist_hbm.at[pl.ds(off, CHUNK)],
                                  d_v, sems.at[1, b]),
        ]

    def out_copy(c, b):
        off = start + c * CHUNK
        return pltpu.make_async_copy(bufs[b][2],
                                     out_hbm.at[pl.ds(off, CHUNK)],
                                     sems.at[2, b])

    for cp in in_copies(0, 0):
        cp.start()

    for c in range(N_CHUNKS):
        b = c % 2
        s_v, d_v, o_v = bufs[b]

        if c + 1 < N_CHUNKS:
            for cp in in_copies(c + 1, (c + 1) % 2):
                cp.start()

        for cp in in_copies(c, b):
            cp.wait()
        if c >= 2:
            out_copy(c - 2, b).wait()

        @plsc.parallel_loop(0, CHUNK // LANES, unroll=8)
        def vec_body(i):
            sl = pl.ds(i * LANES, LANES)
            idx = s_v[0, sl] * 4 + s_v[1, sl]
            o_v[sl] = _pow6(d_v[sl]) + _damp16(idx, damp_tbl)

        out_copy(c, b).start()

    if N_CHUNKS >= 2:
        out_copy(N_CHUNKS - 2, (N_CHUNKS - 2) % 2).wait()
    out_copy(N_CHUNKS - 1, (N_CHUNKS - 1) % 2).wait()

    @pl.when(wid < n_extra_workers)
    def _():
        off = start + N_CHUNKS * CHUNK
        pltpu.sync_copy(species_hbm.at[:, pl.ds(off, TILE)], se_v)
        pltpu.sync_copy(dist_hbm.at[pl.ds(off, TILE)], de_v)

        @plsc.parallel_loop(0, TILE // LANES, unroll=4)
        def vec_body(i):
            sl = pl.ds(i * LANES, LANES)
            idx = se_v[0, sl] * 4 + se_v[1, sl]
            oe_v[sl] = _pow6(de_v[sl]) + _damp16(idx, damp_tbl)
        pltpu.sync_copy(oe_v, out_hbm.at[pl.ds(off, TILE)])


def kernel(species12, distances, cutoff_radii):
    P = distances.shape[0]
    n_tiles = P // TILE
    assert n_tiles * TILE == P
    n_extra = n_tiles - N_WORKERS * N_CHUNKS * CHUNK_TILES
    assert 0 <= n_extra < N_WORKERS

    mesh = plsc.VectorSubcoreMesh(core_axis_name="c", subcore_axis_name="s")
    run = pl.kernel(
        functools.partial(_tec_body, n_extra),
        mesh=mesh,
        out_type=jax.ShapeDtypeStruct((P,), jnp.float32),
        scratch_types=[
            pltpu.VMEM((16,), jnp.float32),
            pltpu.VMEM((2, CHUNK), jnp.int32),
            pltpu.VMEM((2, CHUNK), jnp.int32),
            pltpu.VMEM((CHUNK,), jnp.float32),
            pltpu.VMEM((CHUNK,), jnp.float32),
            pltpu.VMEM((CHUNK,), jnp.float32),
            pltpu.VMEM((CHUNK,), jnp.float32),
            pltpu.VMEM((2, TILE), jnp.int32),
            pltpu.VMEM((TILE,), jnp.float32),
            pltpu.VMEM((TILE,), jnp.float32),
            pltpu.SemaphoreType.DMA((3, 2)),
        ],
    )
    return run(species12.astype(jnp.int32), distances.astype(jnp.float32),
               cutoff_radii.astype(jnp.float32).reshape(16))


# speedup vs baseline: 1.0185x; 1.0185x over previous
import functools

import jax
import jax.numpy as jnp
from jax import lax
from jax.experimental import pallas as pl
from jax.experimental.pallas import tpu as pltpu
from jax.experimental.pallas import tpu_sc as plsc

A1 = 0.3981
A2 = 4.4211
LANES = 16
TILE = 128
N_WORKERS = 32
CHUNK_TILES = 65
CHUNK = CHUNK_TILES * TILE
N_CHUNKS = 6


def _pow6(x):
    x2 = x * x
    return x2 * x2 * x2


def _damp16(idx, damp_tbl):
    return lax.gather(
        damp_tbl, idx[:, None],
        lax.GatherDimensionNumbers(offset_dims=(),
                                   collapsed_slice_dims=(0,),
                                   start_index_map=(0,)),
        slice_sizes=(1,),
        mode=lax.GatherScatterMode.PROMISE_IN_BOUNDS)


def _tec_body(n_extra_workers, species_hbm, dist_hbm, table_hbm, out_hbm,
              tbl_v, sa, sb, da, db, oa, ob, se_v, de_v, oe_v, sems):
    nc = 2
    wid = lax.axis_index("s") * nc + lax.axis_index("c")
    start = (wid * (N_CHUNKS * CHUNK_TILES)
             + jnp.minimum(wid, n_extra_workers)) * TILE
    bufs = [(sa, da, oa), (sb, db, ob)]

    pltpu.sync_copy(table_hbm, tbl_v)
    damp_tbl = _pow6(A1 * tbl_v[...] + A2)

    def in_copies(c, b):
        off = start + c * CHUNK
        s_v, d_v, _ = bufs[b]
        return [
            pltpu.make_async_copy(species_hbm.at[:, pl.ds(off, CHUNK)],
                                  s_v, sems.at[0, b]),
            pltpu.make_async_copy(dist_hbm.at[pl.ds(off, CHUNK)],
                                  d_v, sems.at[1, b]),
        ]

    def out_copy(c, b):
        off = start + c * CHUNK
        return pltpu.make_async_copy(bufs[b][2],
                                     out_hbm.at[pl.ds(off, CHUNK)],
                                     sems.at[2, b])

    for cp in in_copies(0, 0):
        cp.start()

    for c in range(N_CHUNKS):
        b = c % 2
        s_v, d_v, o_v = bufs[b]

        if c + 1 < N_CHUNKS:
            for cp in in_copies(c + 1, (c + 1) % 2):
                cp.start()

        for cp in in_copies(c, b):
            cp.wait()
        if c >= 2:
            out_copy(c - 2, b).wait()

        @plsc.parallel_loop(0, CHUNK // LANES, unroll=8)
        def vec_body(i):
            sl = pl.ds(i * LANES, LANES)
            idx = s_v[0, sl] * 4 + s_v[1, sl]
            o_v[sl] = _pow6(d_v[sl]) + _damp16(idx, damp_tbl)

        out_copy(c, b).start()

    if N_CHUNKS >= 2:
        out_copy(N_CHUNKS - 2, (N_CHUNKS - 2) % 2).wait()
    out_copy(N_CHUNKS - 1, (N_CHUNKS - 1) % 2).wait()

    @pl.when(wid < n_extra_workers)
    def _():
        off = start + N_CHUNKS * CHUNK
        pltpu.sync_copy(species_hbm.at[:, pl.ds(off, TILE)], se_v)
        pltpu.sync_copy(dist_hbm.at[pl.ds(off, TILE)], de_v)

        @plsc.parallel_loop(0, TILE // LANES, unroll=4)
        def vec_body(i):
            sl = pl.ds(i * LANES, LANES)
            idx = se_v[0, sl] * 4 + se_v[1, sl]
            oe_v[sl] = _pow6(de_v[sl]) + _damp16(idx, damp_tbl)
        pltpu.sync_copy(oe_v, out_hbm.at[pl.ds(off, TILE)])


def kernel(species12, distances, cutoff_radii):
    P = distances.shape[0]
    n_tiles = P // TILE
    assert n_tiles * TILE == P
    n_extra = n_tiles - N_WORKERS * N_CHUNKS * CHUNK_TILES
    assert 0 <= n_extra < N_WORKERS

    mesh = plsc.VectorSubcoreMesh(core_axis_name="c", subcore_axis_name="s")
    run = pl.kernel(
        functools.partial(_tec_body, n_extra),
        mesh=mesh,
        out_type=jax.ShapeDtypeStruct((P,), jnp.float32),
        scratch_types=[
            pltpu.VMEM((16,), jnp.float32),
            pltpu.VMEM((2, CHUNK), jnp.int32),
            pltpu.VMEM((2, CHUNK), jnp.int32),
            pltpu.VMEM((CHUNK,), jnp.float32),
            pltpu.VMEM((CHUNK,), jnp.float32),
            pltpu.VMEM((CHUNK,), jnp.float32),
            pltpu.VMEM((CHUNK,), jnp.float32),
            pltpu.VMEM((2, TILE), jnp.int32),
            pltpu.VMEM((TILE,), jnp.float32),
            pltpu.VMEM((TILE,), jnp.float32),
            pltpu.SemaphoreType.DMA((3, 2)),
        ],
    )
    return run(species12.astype(jnp.int32), distances.astype(jnp.float32),
               cutoff_radii.astype(jnp.float32).reshape(16))
